# Initial kernel scaffold; baseline (speedup 1.0000x reference)
import jax
import jax.numpy as jnp
from jax.experimental import pallas as pl


def kernel(x, beta, edge_index):
    raise NotImplementedError("write your pallas kernel here")


# speedup vs baseline: 2.3949x; 2.3949x over previous
import dataclasses
import functools

import jax
import jax.numpy as jnp
from jax import lax
from jax.experimental import pallas as pl
from jax.experimental.pallas import tpu as pltpu
from jax.experimental.pallas import tpu_sc as plsc

N = 10000
D = 128
E = 320000
NC = 2
NS = 16
NW = NC * NS
EP = E // NW
C = 80
NCH = EP // C
G = C // 16
ROWS_PT = 624
REM_ROWS = N - NS * ROWS_PT

_mesh = plsc.VectorSubcoreMesh(core_axis_name="c", subcore_axis_name="s")

_sc_params = pltpu.CompilerParams()
if "needs_layout_passes" in pltpu.CompilerParams.__dataclass_fields__:
    _sc_params = dataclasses.replace(_sc_params, needs_layout_passes=False)


def _normalize_body(x_ref, y_ref):
    xb = x_ref[...]
    n2 = jnp.sum(xb * xb, axis=1, keepdims=True)
    inv = jnp.where(n2 > 0, lax.rsqrt(n2), 0.0)
    y_ref[...] = xb * inv


def _tc_normalize(x):
    return pl.pallas_call(
        _normalize_body,
        out_shape=jax.ShapeDtypeStruct((N, D), jnp.float32),
        grid=(10,),
        in_specs=[pl.BlockSpec((N // 10, D), lambda i: (i, 0))],
        out_specs=pl.BlockSpec((N // 10, D), lambda i: (i, 0)),
    )(x)


@functools.partial(
    pl.kernel,
    mesh=_mesh,
    compiler_params=_sc_params,
    out_type=[
        jax.ShapeDtypeStruct((E,), jnp.float32),
        jax.ShapeDtypeStruct((NW, N), jnp.float32),
    ],
    scratch_types=[
        pltpu.VMEM((EP,), jnp.int32),
        pltpu.VMEM((EP,), jnp.int32),
        pltpu.VMEM((EP,), jnp.float32),
        pltpu.VMEM((N,), jnp.float32),
        pltpu.VMEM((C, D), jnp.float32),
        pltpu.VMEM((C, D), jnp.float32),
        pltpu.VMEM((16,), jnp.float32),
    ],
)
def _sc_edge(y_hbm, row_hbm, col_hbm, betav_hbm, ex_hbm, dpart_hbm,
             rowall, colall, exall, dloc, abuf, bbuf, betabuf):
    cid = lax.axis_index("c")
    sid = lax.axis_index("s")
    wid = cid * NS + sid
    base = wid * EP

    pltpu.sync_copy(row_hbm.at[pl.ds(base, EP)], rowall)
    pltpu.sync_copy(col_hbm.at[pl.ds(base, EP)], colall)
    pltpu.sync_copy(betav_hbm, betabuf)

    @pl.loop(0, N, step=16)
    def _zero(i):
        dloc[pl.ds(i, 16)] = jnp.zeros((16,), jnp.float32)

    @pl.loop(0, NCH)
    def _chunk(ch):
        off = ch * C
        pltpu.sync_copy(y_hbm.at[rowall.at[pl.ds(off, C)]], abuf)
        pltpu.sync_copy(y_hbm.at[colall.at[pl.ds(off, C)]], bbuf)
        betav = betabuf[...]

        @pl.loop(0, G)
        def _group(g):
            e16 = g * 16
            eidx = e16 + lax.iota(jnp.int32, 16)
            acc = jnp.zeros((16,), jnp.float32)
            for j in range(D):
                jv = jnp.full((16,), j, jnp.int32)
                va = plsc.load_gather(abuf, [eidx, jv])
                vb = plsc.load_gather(bbuf, [eidx, jv])
                acc = acc + va * vb
            ex = jnp.exp(acc * betav)
            exall[pl.ds(off + e16, 16)] = ex
            ridx = rowall[pl.ds(off + e16, 16)]
            plsc.addupdate_scatter(dloc, [ridx], ex)

    pltpu.sync_copy(exall, ex_hbm.at[pl.ds(base, EP)])
    pltpu.sync_copy(dloc, dpart_hbm.at[wid])


def _invdenom_body(dp_ref, inv_ref):
    s = jnp.sum(dp_ref[...], axis=0, keepdims=True)
    inv_ref[...] = 1.0 / s


def _tc_invdenom(dpart):
    return pl.pallas_call(
        _invdenom_body,
        out_shape=jax.ShapeDtypeStruct((1, N), jnp.float32),
        grid=(1,),
        in_specs=[pl.BlockSpec((NW, N), lambda i: (0, 0))],
        out_specs=pl.BlockSpec((1, N), lambda i: (0, 0)),
    )(dpart)


@functools.partial(
    pl.kernel,
    mesh=_mesh,
    compiler_params=_sc_params,
    out_type=jax.ShapeDtypeStruct((NC * N, D), jnp.float32),
    scratch_types=[
        pltpu.VMEM((EP,), jnp.int32),
        pltpu.VMEM((EP,), jnp.float32),
        pltpu.VMEM((N,), jnp.float32),
        pltpu.VMEM((C,), jnp.int32),
        pltpu.VMEM((C, D), jnp.float32),
        pltpu.VMEM_SHARED((N, D), jnp.float32),
    ],
)
def _sc_scatter(x_hbm, row_hbm, col_hbm, ex_hbm, invd_hbm, outp_hbm,
                colall, exall, invloc, rowbuf, xbuf, oshared):
    cid = lax.axis_index("c")
    sid = lax.axis_index("s")
    wid = cid * NS + sid
    base = wid * EP
    myrows = sid * ROWS_PT

    pltpu.sync_copy(col_hbm.at[pl.ds(base, EP)], colall)
    pltpu.sync_copy(ex_hbm.at[pl.ds(base, EP)], exall)
    pltpu.sync_copy(invd_hbm.at[0], invloc)

    @pl.loop(0, C)
    def _zrow(i):
        for k in range(D // 16):
            xbuf[i, pl.ds(k * 16, 16)] = jnp.zeros((16,), jnp.float32)

    @pl.loop(0, ROWS_PT // C)
    def _zcopy(k):
        pltpu.sync_copy(xbuf, oshared.at[pl.ds(myrows + k * C, C)])

    _zrem = ROWS_PT - (ROWS_PT // C) * C
    pltpu.sync_copy(xbuf.at[pl.ds(0, _zrem)],
                    oshared.at[pl.ds(myrows + (ROWS_PT // C) * C, _zrem)])

    @pl.when(sid == NS - 1)
    def _ztail():
        pltpu.sync_copy(xbuf.at[pl.ds(0, REM_ROWS)],
                        oshared.at[pl.ds(NS * ROWS_PT, REM_ROWS)])

    plsc.subcore_barrier()

    @pl.loop(0, NCH)
    def _chunk(ch):
        off = ch * C
        pltpu.sync_copy(row_hbm.at[pl.ds(base + off, C)], rowbuf)
        pltpu.sync_copy(x_hbm.at[colall.at[pl.ds(off, C)]], xbuf)

        @pl.loop(0, G)
        def _group(g):
            e16 = g * 16
            eidx = e16 + lax.iota(jnp.int32, 16)
            ridx = rowbuf[pl.ds(e16, 16)]
            w = exall[pl.ds(off + e16, 16)] * plsc.load_gather(invloc, [ridx])
            for j in range(D):
                jv = jnp.full((16,), j, jnp.int32)
                v = plsc.load_gather(xbuf, [eidx, jv]) * w
                plsc.store_scatter(xbuf, [eidx, jv], v)

        pltpu.sync_copy(xbuf, oshared.at[rowbuf], add=True)

    plsc.subcore_barrier()
    pltpu.sync_copy(oshared.at[pl.ds(myrows, ROWS_PT)],
                    outp_hbm.at[pl.ds(cid * N + myrows, ROWS_PT)])

    @pl.when(sid == NS - 1)
    def _dtail():
        pltpu.sync_copy(oshared.at[pl.ds(NS * ROWS_PT, REM_ROWS)],
                        outp_hbm.at[pl.ds(cid * N + NS * ROWS_PT, REM_ROWS)])


def _combine_body(a_ref, b_ref, o_ref):
    o_ref[...] = a_ref[...] + b_ref[...]


def _tc_combine(outp):
    return pl.pallas_call(
        _combine_body,
        out_shape=jax.ShapeDtypeStruct((N, D), jnp.float32),
        grid=(10,),
        in_specs=[
            pl.BlockSpec((N // 10, D), lambda i: (i, 0)),
            pl.BlockSpec((N // 10, D), lambda i: (i + 10, 0)),
        ],
        out_specs=pl.BlockSpec((N // 10, D), lambda i: (i, 0)),
    )(outp, outp)


def kernel(x, beta, edge_index):
    row = edge_index[0].astype(jnp.int32)
    col = edge_index[1].astype(jnp.int32)
    betav = jnp.full((16,), beta[0], jnp.float32)
    y = _tc_normalize(x)
    ex, dpart = _sc_edge(y, row, col, betav)
    invd = _tc_invdenom(dpart)
    outp = _sc_scatter(x, row, col, ex, invd)
    return _tc_combine(outp)


# speedup vs baseline: 2.6477x; 1.1056x over previous
import dataclasses
import functools

import jax
import jax.numpy as jnp
from jax import lax
from jax.experimental import pallas as pl
from jax.experimental.pallas import tpu as pltpu
from jax.experimental.pallas import tpu_sc as plsc

N = 10000
D = 128
E = 320000
NC = 2
NS = 16
NW = NC * NS
EP = E // NW
C = 80
NCH = EP // C
G = C // 16
ROWS_PT = 624
REM_ROWS = N - NS * ROWS_PT

_mesh = plsc.VectorSubcoreMesh(core_axis_name="c", subcore_axis_name="s")

_sc_params = pltpu.CompilerParams()
if "needs_layout_passes" in pltpu.CompilerParams.__dataclass_fields__:
    _sc_params = dataclasses.replace(_sc_params, needs_layout_passes=False)


def _normalize_body(x_ref, y_ref):
    xb = x_ref[...]
    n2 = jnp.sum(xb * xb, axis=1, keepdims=True)
    inv = jnp.where(n2 > 0, lax.rsqrt(n2), 0.0)
    y_ref[...] = xb * inv


def _tc_normalize(x):
    return pl.pallas_call(
        _normalize_body,
        out_shape=jax.ShapeDtypeStruct((N, D), jnp.float32),
        grid=(10,),
        in_specs=[pl.BlockSpec((N // 10, D), lambda i: (i, 0))],
        out_specs=pl.BlockSpec((N // 10, D), lambda i: (i, 0)),
    )(x)


@functools.partial(
    pl.kernel,
    mesh=_mesh,
    compiler_params=_sc_params,
    out_type=[
        jax.ShapeDtypeStruct((E,), jnp.float32),
        jax.ShapeDtypeStruct((NW, N), jnp.float32),
    ],
    scratch_types=[
        pltpu.VMEM((EP,), jnp.int32),
        pltpu.VMEM((EP,), jnp.int32),
        pltpu.VMEM((EP,), jnp.float32),
        pltpu.VMEM((N,), jnp.float32),
        pltpu.VMEM((C, D), jnp.float32),
        pltpu.VMEM((C, D), jnp.float32),
        pltpu.VMEM((C, D), jnp.float32),
        pltpu.VMEM((C, D), jnp.float32),
        pltpu.VMEM((16,), jnp.float32),
        pltpu.SemaphoreType.DMA,
        pltpu.SemaphoreType.DMA,
        pltpu.SemaphoreType.DMA,
        pltpu.SemaphoreType.DMA,
    ],
)
def _sc_edge(y_hbm, row_hbm, col_hbm, betav_hbm, ex_hbm, dpart_hbm,
             rowall, colall, exall, dloc, a0, a1, b0, b1, betabuf,
             sa0, sa1, sb0, sb1):
    cid = lax.axis_index("c")
    sid = lax.axis_index("s")
    wid = cid * NS + sid
    base = wid * EP

    pltpu.sync_copy(row_hbm.at[pl.ds(base, EP)], rowall)
    pltpu.sync_copy(col_hbm.at[pl.ds(base, EP)], colall)
    pltpu.sync_copy(betav_hbm, betabuf)

    @pl.loop(0, N, step=16)
    def _zero(i):
        dloc[pl.ds(i, 16)] = jnp.zeros((16,), jnp.float32)

    def _start(ch, ab, bb, sa, sb):
        off = ch * C
        pltpu.async_copy(y_hbm.at[rowall.at[pl.ds(off, C)]], ab, sa)
        pltpu.async_copy(y_hbm.at[colall.at[pl.ds(off, C)]], bb, sb)

    def _wait(ch, ab, bb, sa, sb):
        off = ch * C
        pltpu.make_async_copy(y_hbm.at[rowall.at[pl.ds(off, C)]], ab, sa).wait()
        pltpu.make_async_copy(y_hbm.at[colall.at[pl.ds(off, C)]], bb, sb).wait()

    def _compute(ch, ab, bb):
        off = ch * C
        betav = betabuf[...]

        @pl.loop(0, G)
        def _group(g):
            e16 = g * 16
            eidx = e16 + lax.iota(jnp.int32, 16)
            acc0 = jnp.zeros((16,), jnp.float32)
            acc1 = jnp.zeros((16,), jnp.float32)
            acc2 = jnp.zeros((16,), jnp.float32)
            acc3 = jnp.zeros((16,), jnp.float32)
            accs = [acc0, acc1, acc2, acc3]
            for j in range(D):
                jv = jnp.full((16,), j, jnp.int32)
                va = plsc.load_gather(ab, [eidx, jv])
                vb = plsc.load_gather(bb, [eidx, jv])
                accs[j % 4] = accs[j % 4] + va * vb
            acc = (accs[0] + accs[1]) + (accs[2] + accs[3])
            ex = jnp.exp(acc * betav)
            exall[pl.ds(off + e16, 16)] = ex
            ridx = rowall[pl.ds(off + e16, 16)]
            plsc.addupdate_scatter(dloc, [ridx], ex)

    _start(0, a0, b0, sa0, sb0)

    @pl.loop(0, NCH)
    def _chunk(ch):
        @pl.when(ch % 2 == 0)
        def _even():
            @pl.when(ch + 1 < NCH)
            def _pf():
                _start(ch + 1, a1, b1, sa1, sb1)
            _wait(ch, a0, b0, sa0, sb0)
            _compute(ch, a0, b0)

        @pl.when(ch % 2 == 1)
        def _odd():
            @pl.when(ch + 1 < NCH)
            def _pf():
                _start(ch + 1, a0, b0, sa0, sb0)
            _wait(ch, a1, b1, sa1, sb1)
            _compute(ch, a1, b1)

    pltpu.sync_copy(exall, ex_hbm.at[pl.ds(base, EP)])
    pltpu.sync_copy(dloc, dpart_hbm.at[wid])


def _invdenom_body(dp_ref, inv_ref):
    s = jnp.sum(dp_ref[...], axis=0, keepdims=True)
    inv_ref[...] = 1.0 / s


def _tc_invdenom(dpart):
    return pl.pallas_call(
        _invdenom_body,
        out_shape=jax.ShapeDtypeStruct((1, N), jnp.float32),
        grid=(1,),
        in_specs=[pl.BlockSpec((NW, N), lambda i: (0, 0))],
        out_specs=pl.BlockSpec((1, N), lambda i: (0, 0)),
    )(dpart)


@functools.partial(
    pl.kernel,
    mesh=_mesh,
    compiler_params=_sc_params,
    out_type=jax.ShapeDtypeStruct((NC * N, D), jnp.float32),
    scratch_types=[
        pltpu.VMEM((EP,), jnp.int32),
        pltpu.VMEM((EP,), jnp.float32),
        pltpu.VMEM((N,), jnp.float32),
        pltpu.VMEM((C,), jnp.int32),
        pltpu.VMEM((C,), jnp.int32),
        pltpu.VMEM((C, D), jnp.float32),
        pltpu.VMEM((C, D), jnp.float32),
        pltpu.VMEM_SHARED((N, D), jnp.float32),
        pltpu.SemaphoreType.DMA,
        pltpu.SemaphoreType.DMA,
        pltpu.SemaphoreType.DMA,
        pltpu.SemaphoreType.DMA,
        pltpu.SemaphoreType.DMA,
        pltpu.SemaphoreType.DMA,
    ],
)
def _sc_scatter(x_hbm, row_hbm, col_hbm, ex_hbm, invd_hbm, outp_hbm,
                colall, exall, invloc, row0, row1, x0, x1, oshared,
                sr0, sr1, sx0, sx1, ss0, ss1):
    cid = lax.axis_index("c")
    sid = lax.axis_index("s")
    wid = cid * NS + sid
    base = wid * EP
    myrows = sid * ROWS_PT

    pltpu.sync_copy(col_hbm.at[pl.ds(base, EP)], colall)
    pltpu.sync_copy(ex_hbm.at[pl.ds(base, EP)], exall)
    pltpu.sync_copy(invd_hbm.at[0], invloc)

    @pl.loop(0, C)
    def _zrow(i):
        for k in range(D // 16):
            x0[i, pl.ds(k * 16, 16)] = jnp.zeros((16,), jnp.float32)

    @pl.loop(0, ROWS_PT // C)
    def _zcopy(k):
        pltpu.sync_copy(x0, oshared.at[pl.ds(myrows + k * C, C)])

    _zrem = ROWS_PT - (ROWS_PT // C) * C
    pltpu.sync_copy(x0.at[pl.ds(0, _zrem)],
                    oshared.at[pl.ds(myrows + (ROWS_PT // C) * C, _zrem)])

    @pl.when(sid == NS - 1)
    def _ztail():
        pltpu.sync_copy(x0.at[pl.ds(0, REM_ROWS)],
                        oshared.at[pl.ds(NS * ROWS_PT, REM_ROWS)])

    plsc.subcore_barrier()

    def _start(ch, rb, xb, sr, sx):
        off = ch * C
        pltpu.async_copy(row_hbm.at[pl.ds(base + off, C)], rb, sr)
        pltpu.async_copy(x_hbm.at[colall.at[pl.ds(off, C)]], xb, sx)

    def _wait(ch, rb, xb, sr, sx):
        off = ch * C
        pltpu.make_async_copy(row_hbm.at[pl.ds(base + off, C)], rb, sr).wait()
        pltpu.make_async_copy(x_hbm.at[colall.at[pl.ds(off, C)]], xb, sx).wait()

    def _compute(ch, rb, xb):
        off = ch * C

        @pl.loop(0, G)
        def _group(g):
            e16 = g * 16
            eidx = e16 + lax.iota(jnp.int32, 16)
            ridx = rb[pl.ds(e16, 16)]
            w = exall[pl.ds(off + e16, 16)] * plsc.load_gather(invloc, [ridx])
            for j in range(D):
                jv = jnp.full((16,), j, jnp.int32)
                v = plsc.load_gather(xb, [eidx, jv]) * w
                plsc.store_scatter(xb, [eidx, jv], v)

    def _scat_start(rb, xb, ss):
        pltpu.async_copy(xb, oshared.at[rb], sem=ss, add=True)

    def _scat_wait(rb, xb, ss):
        pltpu.make_async_copy(xb, oshared.at[rb], ss).wait()

    _start(0, row0, x0, sr0, sx0)

    @pl.loop(0, NCH)
    def _chunk(ch):
        @pl.when(ch % 2 == 0)
        def _even():
            @pl.when(ch >= 1)
            def _ws():
                _scat_wait(row1, x1, ss1)

            @pl.when(ch + 1 < NCH)
            def _pf():
                _start(ch + 1, row1, x1, sr1, sx1)
            _wait(ch, row0, x0, sr0, sx0)
            _compute(ch, row0, x0)
            _scat_start(row0, x0, ss0)

        @pl.when(ch % 2 == 1)
        def _odd():
            _scat_wait(row0, x0, ss0)

            @pl.when(ch + 1 < NCH)
            def _pf():
                _start(ch + 1, row0, x0, sr0, sx0)
            _wait(ch, row1, x1, sr1, sx1)
            _compute(ch, row1, x1)
            _scat_start(row1, x1, ss1)

    _scat_wait(row0, x0, ss0)
    plsc.subcore_barrier()
    pltpu.sync_copy(oshared.at[pl.ds(myrows, ROWS_PT)],
                    outp_hbm.at[pl.ds(cid * N + myrows, ROWS_PT)])

    @pl.when(sid == NS - 1)
    def _dtail():
        pltpu.sync_copy(oshared.at[pl.ds(NS * ROWS_PT, REM_ROWS)],
                        outp_hbm.at[pl.ds(cid * N + NS * ROWS_PT, REM_ROWS)])


def _combine_body(a_ref, b_ref, o_ref):
    o_ref[...] = a_ref[...] + b_ref[...]


def _tc_combine(outp):
    return pl.pallas_call(
        _combine_body,
        out_shape=jax.ShapeDtypeStruct((N, D), jnp.float32),
        grid=(10,),
        in_specs=[
            pl.BlockSpec((N // 10, D), lambda i: (i, 0)),
            pl.BlockSpec((N // 10, D), lambda i: (i + 10, 0)),
        ],
        out_specs=pl.BlockSpec((N // 10, D), lambda i: (i, 0)),
    )(outp, outp)


def kernel(x, beta, edge_index):
    row = edge_index[0].astype(jnp.int32)
    col = edge_index[1].astype(jnp.int32)
    betav = jnp.full((16,), beta[0], jnp.float32)
    y = _tc_normalize(x)
    ex, dpart = _sc_edge(y, row, col, betav)
    invd = _tc_invdenom(dpart)
    outp = _sc_scatter(x, row, col, ex, invd)
    return _tc_combine(outp)


# speedup vs baseline: 6.4022x; 2.4180x over previous
import dataclasses
import functools

import jax
import jax.numpy as jnp
from jax import lax
from jax.experimental import pallas as pl
from jax.experimental.pallas import tpu as pltpu
from jax.experimental.pallas import tpu_sc as plsc

N = 10000
D = 128
E = 320000
NC = 2
NS = 16
NW = NC * NS
CPT = D // NW
EP = E // NW
SCH_A = 16000
SCH_B = 8000

_mesh = plsc.VectorSubcoreMesh(core_axis_name="c", subcore_axis_name="s")

_sc_params = pltpu.CompilerParams()
if "needs_layout_passes" in pltpu.CompilerParams.__dataclass_fields__:
    _sc_params = dataclasses.replace(_sc_params, needs_layout_passes=False)


def _normalize_body(x_ref, y_ref):
    xb = x_ref[...]
    n2 = jnp.sum(xb * xb, axis=1, keepdims=True)
    inv = jnp.where(n2 > 0, lax.rsqrt(n2), 0.0)
    y_ref[...] = xb * inv


def _tc_normalize(x):
    return pl.pallas_call(
        _normalize_body,
        out_shape=jax.ShapeDtypeStruct((N, D), jnp.float32),
        grid=(10,),
        in_specs=[pl.BlockSpec((N // 10, D), lambda i: (i, 0))],
        out_specs=pl.BlockSpec((N // 10, D), lambda i: (i, 0)),
    )(x)


@functools.partial(
    pl.kernel,
    mesh=_mesh,
    compiler_params=_sc_params,
    out_type=jax.ShapeDtypeStruct((NW * E,), jnp.float32),
    scratch_types=[
        pltpu.VMEM((CPT, N), jnp.float32),
        pltpu.VMEM((SCH_A,), jnp.int32),
        pltpu.VMEM((SCH_A,), jnp.int32),
        pltpu.VMEM((SCH_A,), jnp.float32),
    ],
)
def _sc_dots(yt_hbm, row_hbm, col_hbm, part_hbm, ytloc, rowb, colb, pbuf):
    cid = lax.axis_index("c")
    sid = lax.axis_index("s")
    wid = cid * NS + sid

    pltpu.sync_copy(yt_hbm.at[pl.ds(wid * CPT, CPT)], ytloc)

    @pl.loop(0, E // SCH_A)
    def _sch(sc):
        off = sc * SCH_A
        pltpu.sync_copy(row_hbm.at[pl.ds(off, SCH_A)], rowb)
        pltpu.sync_copy(col_hbm.at[pl.ds(off, SCH_A)], colb)

        @pl.loop(0, SCH_A // 16)
        def _group(g):
            e16 = g * 16
            ridx = rowb[pl.ds(e16, 16)]
            cidx = colb[pl.ds(e16, 16)]
            acc = jnp.zeros((16,), jnp.float32)
            for j in range(CPT):
                jv = jnp.full((16,), j, jnp.int32)
                va = plsc.load_gather(ytloc, [jv, ridx])
                vb = plsc.load_gather(ytloc, [jv, cidx])
                acc = acc + va * vb
            pbuf[pl.ds(e16, 16)] = acc

        pltpu.sync_copy(pbuf, part_hbm.at[pl.ds(wid * E + off, SCH_A)])


def _exp_body(b_ref, p_ref, ex_ref):
    s = jnp.sum(p_ref[...], axis=0, keepdims=True)
    ex_ref[...] = jnp.exp(b_ref[0, 0] * s)


def _tc_exp(part, beta2d):
    nblk = 20
    return pl.pallas_call(
        _exp_body,
        out_shape=jax.ShapeDtypeStruct((1, E), jnp.float32),
        grid=(nblk,),
        in_specs=[
            pl.BlockSpec((1, 1), lambda i: (0, 0)),
            pl.BlockSpec((NW, E // nblk), lambda i: (0, i)),
        ],
        out_specs=pl.BlockSpec((1, E // nblk), lambda i: (0, i)),
    )(beta2d, part)


@functools.partial(
    pl.kernel,
    mesh=_mesh,
    compiler_params=_sc_params,
    out_type=jax.ShapeDtypeStruct((NW, N), jnp.float32),
    scratch_types=[
        pltpu.VMEM((EP,), jnp.int32),
        pltpu.VMEM((EP,), jnp.float32),
        pltpu.VMEM((N,), jnp.float32),
    ],
)
def _sc_denom(row_hbm, ex_hbm, dpart_hbm, rowb, exb, dloc):
    cid = lax.axis_index("c")
    sid = lax.axis_index("s")
    wid = cid * NS + sid
    base = wid * EP

    pltpu.sync_copy(row_hbm.at[pl.ds(base, EP)], rowb)
    pltpu.sync_copy(ex_hbm.at[pl.ds(base, EP)], exb)

    @pl.loop(0, N, step=16)
    def _zero(i):
        dloc[pl.ds(i, 16)] = jnp.zeros((16,), jnp.float32)

    @pl.loop(0, EP // 16)
    def _group(g):
        e16 = g * 16
        ridx = rowb[pl.ds(e16, 16)]
        ex = exb[pl.ds(e16, 16)]
        plsc.addupdate_scatter(dloc, [ridx], ex)

    pltpu.sync_copy(dloc, dpart_hbm.at[wid])


def _invdenom_body(dp_ref, inv_ref):
    s = jnp.sum(dp_ref[...], axis=0, keepdims=True)
    inv_ref[...] = 1.0 / s


def _tc_invdenom(dpart):
    return pl.pallas_call(
        _invdenom_body,
        out_shape=jax.ShapeDtypeStruct((1, N), jnp.float32),
        grid=(1,),
        in_specs=[pl.BlockSpec((NW, N), lambda i: (0, 0))],
        out_specs=pl.BlockSpec((1, N), lambda i: (0, 0)),
    )(dpart)


@functools.partial(
    pl.kernel,
    mesh=_mesh,
    compiler_params=_sc_params,
    out_type=jax.ShapeDtypeStruct((D, N), jnp.float32),
    scratch_types=[
        pltpu.VMEM((CPT, N), jnp.float32),
        pltpu.VMEM((CPT, N), jnp.float32),
        pltpu.VMEM((N,), jnp.float32),
        pltpu.VMEM((SCH_B,), jnp.int32),
        pltpu.VMEM((SCH_B,), jnp.int32),
        pltpu.VMEM((SCH_B,), jnp.float32),
    ],
)
def _sc_scatter(xt_hbm, row_hbm, col_hbm, ex_hbm, invd_hbm, outt_hbm,
                xtloc, otloc, invloc, rowb, colb, exb):
    cid = lax.axis_index("c")
    sid = lax.axis_index("s")
    wid = cid * NS + sid

    pltpu.sync_copy(xt_hbm.at[pl.ds(wid * CPT, CPT)], xtloc)
    pltpu.sync_copy(invd_hbm.at[0], invloc)

    @pl.loop(0, N, step=16)
    def _zero(i):
        for j in range(CPT):
            otloc[j, pl.ds(i, 16)] = jnp.zeros((16,), jnp.float32)

    @pl.loop(0, E // SCH_B)
    def _sch(sc):
        off = sc * SCH_B
        pltpu.sync_copy(row_hbm.at[pl.ds(off, SCH_B)], rowb)
        pltpu.sync_copy(col_hbm.at[pl.ds(off, SCH_B)], colb)
        pltpu.sync_copy(ex_hbm.at[pl.ds(off, SCH_B)], exb)

        @pl.loop(0, SCH_B // 16)
        def _group(g):
            e16 = g * 16
            ridx = rowb[pl.ds(e16, 16)]
            cidx = colb[pl.ds(e16, 16)]
            w = exb[pl.ds(e16, 16)] * plsc.load_gather(invloc, [ridx])
            for j in range(CPT):
                jv = jnp.full((16,), j, jnp.int32)
                v = plsc.load_gather(xtloc, [jv, cidx]) * w
                plsc.addupdate_scatter(otloc, [jv, ridx], v)

    pltpu.sync_copy(otloc, outt_hbm.at[pl.ds(wid * CPT, CPT)])


def kernel(x, beta, edge_index):
    row = edge_index[0].astype(jnp.int32)
    col = edge_index[1].astype(jnp.int32)
    beta2d = beta.reshape(1, 1).astype(jnp.float32)
    y = _tc_normalize(x)
    yt = y.T
    xt = x.T
    part = _sc_dots(yt, row, col)
    ex2d = _tc_exp(part.reshape(NW, E), beta2d)
    ex = ex2d.reshape(E)
    dpart = _sc_denom(row, ex)
    invd = _tc_invdenom(dpart)
    outt = _sc_scatter(xt, row, col, ex, invd)
    return outt.T


# speedup vs baseline: 12.2968x; 1.9207x over previous
import dataclasses
import functools

import jax
import jax.numpy as jnp
from jax import lax
from jax.experimental import pallas as pl
from jax.experimental.pallas import tpu as pltpu
from jax.experimental.pallas import tpu_sc as plsc

N = 10000
D = 128
E = 320000
NC = 2
NS = 16
NW = NC * NS
CPT = D // NW
CPT_A = D // NS
EH = E // NC
EP = E // NW
SCH_A = 8000
SCH_B = 8000

_mesh = plsc.VectorSubcoreMesh(core_axis_name="c", subcore_axis_name="s")

_sc_params = pltpu.CompilerParams()
if "needs_layout_passes" in pltpu.CompilerParams.__dataclass_fields__:
    _sc_params = dataclasses.replace(_sc_params, needs_layout_passes=False)


def _normalize_body(x_ref, y_ref):
    xb = x_ref[...]
    n2 = jnp.sum(xb * xb, axis=1, keepdims=True)
    inv = jnp.where(n2 > 0, lax.rsqrt(n2), 0.0)
    y_ref[...] = xb * inv


def _tc_normalize(x):
    return pl.pallas_call(
        _normalize_body,
        out_shape=jax.ShapeDtypeStruct((N, D), jnp.float32),
        grid=(10,),
        in_specs=[pl.BlockSpec((N // 10, D), lambda i: (i, 0))],
        out_specs=pl.BlockSpec((N // 10, D), lambda i: (i, 0)),
    )(x)


@functools.partial(
    pl.kernel,
    mesh=_mesh,
    compiler_params=_sc_params,
    out_type=jax.ShapeDtypeStruct((NS * E,), jnp.float32),
    scratch_types=[
        pltpu.VMEM((CPT_A, N), jnp.float32),
        pltpu.VMEM((SCH_A,), jnp.int32),
        pltpu.VMEM((SCH_A,), jnp.int32),
        pltpu.VMEM((SCH_A,), jnp.float32),
    ],
)
def _sc_dots(yt_hbm, row_hbm, col_hbm, part_hbm, ytloc, rowb, colb, pbuf):
    cid = lax.axis_index("c")
    sid = lax.axis_index("s")
    ebase = cid * EH

    pltpu.sync_copy(yt_hbm.at[pl.ds(sid * CPT_A, CPT_A)], ytloc)

    @pl.loop(0, EH // SCH_A)
    def _sch(sc):
        off = sc * SCH_A
        pltpu.sync_copy(row_hbm.at[pl.ds(ebase + off, SCH_A)], rowb)
        pltpu.sync_copy(col_hbm.at[pl.ds(ebase + off, SCH_A)], colb)

        @plsc.parallel_loop(0, SCH_A // 16, unroll=4)
        def _group(g):
            e16 = g * 16
            ridx = rowb[pl.ds(e16, 16)]
            cidx = colb[pl.ds(e16, 16)]
            acc = jnp.zeros((16,), jnp.float32)
            for j in range(CPT_A):
                jv = jnp.full((16,), j, jnp.int32)
                va = plsc.load_gather(ytloc, [jv, ridx])
                vb = plsc.load_gather(ytloc, [jv, cidx])
                acc = acc + va * vb
            pbuf[pl.ds(e16, 16)] = acc

        pltpu.sync_copy(pbuf, part_hbm.at[pl.ds(sid * E + ebase + off, SCH_A)])


def _exp_body(b_ref, p_ref, ex_ref):
    s = jnp.sum(p_ref[...], axis=0, keepdims=True)
    ex_ref[...] = jnp.exp(b_ref[0, 0] * s)


def _tc_exp(part, beta2d):
    nblk = 20
    return pl.pallas_call(
        _exp_body,
        out_shape=jax.ShapeDtypeStruct((1, E), jnp.float32),
        grid=(nblk,),
        in_specs=[
            pl.BlockSpec((1, 1), lambda i: (0, 0)),
            pl.BlockSpec((NS, E // nblk), lambda i: (0, i)),
        ],
        out_specs=pl.BlockSpec((1, E // nblk), lambda i: (0, i)),
    )(beta2d, part)


@functools.partial(
    pl.kernel,
    mesh=_mesh,
    compiler_params=_sc_params,
    out_type=jax.ShapeDtypeStruct((NW, N), jnp.float32),
    scratch_types=[
        pltpu.VMEM((EP,), jnp.int32),
        pltpu.VMEM((EP,), jnp.float32),
        pltpu.VMEM((N,), jnp.float32),
    ],
)
def _sc_denom(row_hbm, ex_hbm, dpart_hbm, rowb, exb, dloc):
    cid = lax.axis_index("c")
    sid = lax.axis_index("s")
    wid = cid * NS + sid
    base = wid * EP

    pltpu.sync_copy(row_hbm.at[pl.ds(base, EP)], rowb)
    pltpu.sync_copy(ex_hbm.at[pl.ds(base, EP)], exb)

    @pl.loop(0, N, step=16)
    def _zero(i):
        dloc[pl.ds(i, 16)] = jnp.zeros((16,), jnp.float32)

    @plsc.parallel_loop(0, EP // 16, unroll=4)
    def _group(g):
        e16 = g * 16
        ridx = rowb[pl.ds(e16, 16)]
        ex = exb[pl.ds(e16, 16)]
        plsc.addupdate_scatter(dloc, [ridx], ex)

    pltpu.sync_copy(dloc, dpart_hbm.at[wid])


def _invdenom_body(dp_ref, inv_ref):
    s = jnp.sum(dp_ref[...], axis=0, keepdims=True)
    inv_ref[...] = 1.0 / s


def _tc_invdenom(dpart):
    return pl.pallas_call(
        _invdenom_body,
        out_shape=jax.ShapeDtypeStruct((1, N), jnp.float32),
        grid=(1,),
        in_specs=[pl.BlockSpec((NW, N), lambda i: (0, 0))],
        out_specs=pl.BlockSpec((1, N), lambda i: (0, 0)),
    )(dpart)


@functools.partial(
    pl.kernel,
    mesh=_mesh,
    compiler_params=_sc_params,
    out_type=jax.ShapeDtypeStruct((D, N), jnp.float32),
    scratch_types=[
        pltpu.VMEM((CPT, N), jnp.float32),
        pltpu.VMEM((CPT, N), jnp.float32),
        pltpu.VMEM((N,), jnp.float32),
        pltpu.VMEM((SCH_B,), jnp.int32),
        pltpu.VMEM((SCH_B,), jnp.int32),
        pltpu.VMEM((SCH_B,), jnp.float32),
    ],
)
def _sc_scatter(xt_hbm, row_hbm, col_hbm, ex_hbm, invd_hbm, outt_hbm,
                xtloc, otloc, invloc, rowb, colb, exb):
    cid = lax.axis_index("c")
    sid = lax.axis_index("s")
    wid = cid * NS + sid

    pltpu.sync_copy(xt_hbm.at[pl.ds(wid * CPT, CPT)], xtloc)
    pltpu.sync_copy(invd_hbm.at[0], invloc)

    @pl.loop(0, N, step=16)
    def _zero(i):
        for j in range(CPT):
            otloc[j, pl.ds(i, 16)] = jnp.zeros((16,), jnp.float32)

    @pl.loop(0, E // SCH_B)
    def _sch(sc):
        off = sc * SCH_B
        pltpu.sync_copy(row_hbm.at[pl.ds(off, SCH_B)], rowb)
        pltpu.sync_copy(col_hbm.at[pl.ds(off, SCH_B)], colb)
        pltpu.sync_copy(ex_hbm.at[pl.ds(off, SCH_B)], exb)

        @plsc.parallel_loop(0, SCH_B // 16, unroll=4)
        def _group(g):
            e16 = g * 16
            ridx = rowb[pl.ds(e16, 16)]
            cidx = colb[pl.ds(e16, 16)]
            w = exb[pl.ds(e16, 16)] * plsc.load_gather(invloc, [ridx])
            for j in range(CPT):
                jv = jnp.full((16,), j, jnp.int32)
                v = plsc.load_gather(xtloc, [jv, cidx]) * w
                plsc.addupdate_scatter(otloc, [jv, ridx], v)

    pltpu.sync_copy(otloc, outt_hbm.at[pl.ds(wid * CPT, CPT)])


def kernel(x, beta, edge_index):
    row = edge_index[0].astype(jnp.int32)
    col = edge_index[1].astype(jnp.int32)
    beta2d = beta.reshape(1, 1).astype(jnp.float32)
    y = _tc_normalize(x)
    yt = y.T
    xt = x.T
    part = _sc_dots(yt, row, col)
    ex2d = _tc_exp(part.reshape(NS, E), beta2d)
    ex = ex2d.reshape(E)
    dpart = _sc_denom(row, ex)
    invd = _tc_invdenom(dpart)
    outt = _sc_scatter(xt, row, col, ex, invd)
    return outt.T


# speedup vs baseline: 16.1995x; 1.3174x over previous
import dataclasses
import functools

import jax
import jax.numpy as jnp
from jax import lax
from jax.experimental import pallas as pl
from jax.experimental.pallas import tpu as pltpu
from jax.experimental.pallas import tpu_sc as plsc

N = 10000
D = 128
E = 320000
NC = 2
NS = 16
NW = NC * NS
CPT = D // NW
CPT_A = D // NS
EH = E // NC
EP = E // NW
SCH_A = 4000
SCH_B = 4000

_mesh = plsc.VectorSubcoreMesh(core_axis_name="c", subcore_axis_name="s")

_sc_params = pltpu.CompilerParams()
if "needs_layout_passes" in pltpu.CompilerParams.__dataclass_fields__:
    _sc_params = dataclasses.replace(_sc_params, needs_layout_passes=False)


def _normalize_body(x_ref, y_ref):
    xb = x_ref[...]
    n2 = jnp.sum(xb * xb, axis=1, keepdims=True)
    inv = jnp.where(n2 > 0, lax.rsqrt(n2), 0.0)
    y_ref[...] = xb * inv


def _tc_normalize(x):
    return pl.pallas_call(
        _normalize_body,
        out_shape=jax.ShapeDtypeStruct((N, D), jnp.float32),
        grid=(10,),
        in_specs=[pl.BlockSpec((N // 10, D), lambda i: (i, 0))],
        out_specs=pl.BlockSpec((N // 10, D), lambda i: (i, 0)),
    )(x)


@functools.partial(
    pl.kernel,
    mesh=_mesh,
    compiler_params=_sc_params,
    out_type=jax.ShapeDtypeStruct((NS * E,), jnp.float32),
    scratch_types=[
        pltpu.VMEM((CPT_A, N), jnp.float32),
        pltpu.VMEM((SCH_A,), jnp.int32),
        pltpu.VMEM((SCH_A,), jnp.int32),
        pltpu.VMEM((SCH_A,), jnp.int32),
        pltpu.VMEM((SCH_A,), jnp.int32),
        pltpu.VMEM((SCH_A,), jnp.float32),
        pltpu.VMEM((SCH_A,), jnp.float32),
        pltpu.SemaphoreType.DMA,
        pltpu.SemaphoreType.DMA,
        pltpu.SemaphoreType.DMA,
        pltpu.SemaphoreType.DMA,
        pltpu.SemaphoreType.DMA,
        pltpu.SemaphoreType.DMA,
    ],
)
def _sc_dots(yt_hbm, row_hbm, col_hbm, part_hbm, ytloc,
             rowb0, rowb1, colb0, colb1, pbuf0, pbuf1,
             sr0, sr1, sc0, sc1, sp0, sp1):
    cid = lax.axis_index("c")
    sid = lax.axis_index("s")
    ebase = cid * EH
    nsch = EH // SCH_A

    pltpu.sync_copy(yt_hbm.at[pl.ds(sid * CPT_A, CPT_A)], ytloc)

    def _in_copies(sc, rb, cb, sr, scm):
        off = ebase + sc * SCH_A
        return (pltpu.make_async_copy(row_hbm.at[pl.ds(off, SCH_A)], rb, sr),
                pltpu.make_async_copy(col_hbm.at[pl.ds(off, SCH_A)], cb, scm))

    def _out_copy(sc, pb, sp):
        off = sc * SCH_A
        return pltpu.make_async_copy(
            pb, part_hbm.at[pl.ds(sid * E + ebase + off, SCH_A)], sp)

    def _start_in(sc, rb, cb, sr, scm):
        off = ebase + sc * SCH_A
        pltpu.async_copy(row_hbm.at[pl.ds(off, SCH_A)], rb, sr)
        pltpu.async_copy(col_hbm.at[pl.ds(off, SCH_A)], cb, scm)

    def _compute(rb, cb, pb):
        @plsc.parallel_loop(0, SCH_A // 16, unroll=4)
        def _group(g):
            e16 = g * 16
            ridx = rb[pl.ds(e16, 16)]
            cidx = cb[pl.ds(e16, 16)]
            acc = jnp.zeros((16,), jnp.float32)
            for j in range(CPT_A):
                jv = jnp.full((16,), j, jnp.int32)
                va = plsc.load_gather(ytloc, [jv, ridx])
                vb = plsc.load_gather(ytloc, [jv, cidx])
                acc = acc + va * vb
            pb[pl.ds(e16, 16)] = acc

    _start_in(0, rowb0, colb0, sr0, sc0)

    @pl.loop(0, nsch)
    def _sch(sc):
        @pl.when(sc % 2 == 0)
        def _even():
            @pl.when(sc + 1 < nsch)
            def _pf():
                _start_in(sc + 1, rowb1, colb1, sr1, sc1)
            for c in _in_copies(sc, rowb0, colb0, sr0, sc0):
                c.wait()

            @pl.when(sc >= 2)
            def _wo():
                _out_copy(sc - 2, pbuf0, sp0).wait()
            _compute(rowb0, colb0, pbuf0)
            pltpu.async_copy(
                pbuf0, part_hbm.at[pl.ds(sid * E + ebase + sc * SCH_A, SCH_A)],
                sp0)

        @pl.when(sc % 2 == 1)
        def _odd():
            @pl.when(sc + 1 < nsch)
            def _pf():
                _start_in(sc + 1, rowb0, colb0, sr0, sc0)
            for c in _in_copies(sc, rowb1, colb1, sr1, sc1):
                c.wait()

            @pl.when(sc >= 2)
            def _wo():
                _out_copy(sc - 2, pbuf1, sp1).wait()
            _compute(rowb1, colb1, pbuf1)
            pltpu.async_copy(
                pbuf1, part_hbm.at[pl.ds(sid * E + ebase + sc * SCH_A, SCH_A)],
                sp1)

    _out_copy(nsch - 2, pbuf0, sp0).wait()
    _out_copy(nsch - 1, pbuf1, sp1).wait()


def _exp_body(b_ref, p_ref, ex_ref):
    s = jnp.sum(p_ref[...], axis=0, keepdims=True)
    ex_ref[...] = jnp.exp(b_ref[0, 0] * s)


def _tc_exp(part, beta2d):
    nblk = 20
    return pl.pallas_call(
        _exp_body,
        out_shape=jax.ShapeDtypeStruct((1, E), jnp.float32),
        grid=(nblk,),
        in_specs=[
            pl.BlockSpec((1, 1), lambda i: (0, 0)),
            pl.BlockSpec((NS, E // nblk), lambda i: (0, i)),
        ],
        out_specs=pl.BlockSpec((1, E // nblk), lambda i: (0, i)),
    )(beta2d, part)


@functools.partial(
    pl.kernel,
    mesh=_mesh,
    compiler_params=_sc_params,
    out_type=jax.ShapeDtypeStruct((NW, N), jnp.float32),
    scratch_types=[
        pltpu.VMEM((EP,), jnp.int32),
        pltpu.VMEM((EP,), jnp.float32),
        pltpu.VMEM((N,), jnp.float32),
    ],
)
def _sc_denom(row_hbm, ex_hbm, dpart_hbm, rowb, exb, dloc):
    cid = lax.axis_index("c")
    sid = lax.axis_index("s")
    wid = cid * NS + sid
    base = wid * EP

    pltpu.sync_copy(row_hbm.at[pl.ds(base, EP)], rowb)
    pltpu.sync_copy(ex_hbm.at[pl.ds(base, EP)], exb)

    @pl.loop(0, N, step=16)
    def _zero(i):
        dloc[pl.ds(i, 16)] = jnp.zeros((16,), jnp.float32)

    @plsc.parallel_loop(0, EP // 16, unroll=4)
    def _group(g):
        e16 = g * 16
        ridx = rowb[pl.ds(e16, 16)]
        ex = exb[pl.ds(e16, 16)]
        plsc.addupdate_scatter(dloc, [ridx], ex)

    pltpu.sync_copy(dloc, dpart_hbm.at[wid])


def _invdenom_body(dp_ref, inv_ref):
    s = jnp.sum(dp_ref[...], axis=0, keepdims=True)
    inv_ref[...] = 1.0 / s


def _tc_invdenom(dpart):
    return pl.pallas_call(
        _invdenom_body,
        out_shape=jax.ShapeDtypeStruct((1, N), jnp.float32),
        grid=(1,),
        in_specs=[pl.BlockSpec((NW, N), lambda i: (0, 0))],
        out_specs=pl.BlockSpec((1, N), lambda i: (0, 0)),
    )(dpart)


@functools.partial(
    pl.kernel,
    mesh=_mesh,
    compiler_params=_sc_params,
    out_type=jax.ShapeDtypeStruct((D, N), jnp.float32),
    scratch_types=[
        pltpu.VMEM((CPT, N), jnp.float32),
        pltpu.VMEM((CPT, N), jnp.float32),
        pltpu.VMEM((N,), jnp.float32),
        pltpu.VMEM((SCH_B,), jnp.int32),
        pltpu.VMEM((SCH_B,), jnp.int32),
        pltpu.VMEM((SCH_B,), jnp.int32),
        pltpu.VMEM((SCH_B,), jnp.int32),
        pltpu.VMEM((SCH_B,), jnp.float32),
        pltpu.VMEM((SCH_B,), jnp.float32),
        pltpu.SemaphoreType.DMA,
        pltpu.SemaphoreType.DMA,
        pltpu.SemaphoreType.DMA,
        pltpu.SemaphoreType.DMA,
        pltpu.SemaphoreType.DMA,
        pltpu.SemaphoreType.DMA,
    ],
)
def _sc_scatter(xt_hbm, row_hbm, col_hbm, ex_hbm, invd_hbm, outt_hbm,
                xtloc, otloc, invloc, rowb0, rowb1, colb0, colb1, exb0, exb1,
                sr0, sr1, sc0, sc1, se0, se1):
    cid = lax.axis_index("c")
    sid = lax.axis_index("s")
    wid = cid * NS + sid
    nsch = E // SCH_B

    pltpu.sync_copy(xt_hbm.at[pl.ds(wid * CPT, CPT)], xtloc)
    pltpu.sync_copy(invd_hbm.at[0], invloc)

    @pl.loop(0, N, step=16)
    def _zero(i):
        for j in range(CPT):
            otloc[j, pl.ds(i, 16)] = jnp.zeros((16,), jnp.float32)

    def _in_copies(sc, rb, cb, eb, sr, scm, se):
        off = sc * SCH_B
        return (pltpu.make_async_copy(row_hbm.at[pl.ds(off, SCH_B)], rb, sr),
                pltpu.make_async_copy(col_hbm.at[pl.ds(off, SCH_B)], cb, scm),
                pltpu.make_async_copy(ex_hbm.at[pl.ds(off, SCH_B)], eb, se))

    def _start_in(sc, rb, cb, eb, sr, scm, se):
        off = sc * SCH_B
        pltpu.async_copy(row_hbm.at[pl.ds(off, SCH_B)], rb, sr)
        pltpu.async_copy(col_hbm.at[pl.ds(off, SCH_B)], cb, scm)
        pltpu.async_copy(ex_hbm.at[pl.ds(off, SCH_B)], eb, se)

    def _compute(rb, cb, eb):
        @plsc.parallel_loop(0, SCH_B // 16, unroll=4)
        def _group(g):
            e16 = g * 16
            ridx = rb[pl.ds(e16, 16)]
            cidx = cb[pl.ds(e16, 16)]
            w = eb[pl.ds(e16, 16)] * plsc.load_gather(invloc, [ridx])
            for j in range(CPT):
                jv = jnp.full((16,), j, jnp.int32)
                v = plsc.load_gather(xtloc, [jv, cidx]) * w
                plsc.addupdate_scatter(otloc, [jv, ridx], v)

    _start_in(0, rowb0, colb0, exb0, sr0, sc0, se0)

    @pl.loop(0, nsch)
    def _sch(sc):
        @pl.when(sc % 2 == 0)
        def _even():
            @pl.when(sc + 1 < nsch)
            def _pf():
                _start_in(sc + 1, rowb1, colb1, exb1, sr1, sc1, se1)
            for c in _in_copies(sc, rowb0, colb0, exb0, sr0, sc0, se0):
                c.wait()
            _compute(rowb0, colb0, exb0)

        @pl.when(sc % 2 == 1)
        def _odd():
            @pl.when(sc + 1 < nsch)
            def _pf():
                _start_in(sc + 1, rowb0, colb0, exb0, sr0, sc0, se0)
            for c in _in_copies(sc, rowb1, colb1, exb1, sr1, sc1, se1):
                c.wait()
            _compute(rowb1, colb1, exb1)

    pltpu.sync_copy(otloc, outt_hbm.at[pl.ds(wid * CPT, CPT)])


def kernel(x, beta, edge_index):
    row = edge_index[0].astype(jnp.int32)
    col = edge_index[1].astype(jnp.int32)
    beta2d = beta.reshape(1, 1).astype(jnp.float32)
    y = _tc_normalize(x)
    yt = y.T
    xt = x.T
    part = _sc_dots(yt, row, col)
    ex2d = _tc_exp(part.reshape(NS, E), beta2d)
    ex = ex2d.reshape(E)
    dpart = _sc_denom(row, ex)
    invd = _tc_invdenom(dpart)
    outt = _sc_scatter(xt, row, col, ex, invd)
    return outt.T
